# Initial kernel scaffold; baseline (speedup 1.0000x reference)
import jax
import jax.numpy as jnp
from jax.experimental import pallas as pl


def kernel(inputs, Hs, mask_item, item, embedding, a1, a2):
    raise NotImplementedError("write your pallas kernel here")


# speedup vs baseline: 1.1632x; 1.1632x over previous
import functools

import jax
import jax.numpy as jnp
from jax import lax
from jax.experimental import pallas as pl
from jax.experimental.pallas import tpu as pltpu
from jax.experimental.pallas import tpu_sc as plsc

DIM = 64
B = 1024
L = 50
ALPHA = 0.2
NEG = -1e9

NC = 2
NS = 16
NW = NC * NS
NIDX = 2 * B * L
BPW = NIDX // NW
CH = 1600
HALF = B * L


def _gather_body(table, idx, out_h, out_i, idx_v, rows_v, sem):
    wid = lax.axis_index("s") * NC + lax.axis_index("c")
    base = wid * BPW
    pltpu.sync_copy(idx.at[pl.ds(base, BPW)], idx_v)
    for c in range(BPW // CH):
        pltpu.async_copy(table.at[idx_v.at[pl.ds(c * CH, CH)]], rows_v, sem).wait()
        off = base + c * CH

        @pl.when(wid < NW // 2)
        def _():
            pltpu.sync_copy(rows_v, out_h.at[pl.ds(off, CH)])

        @pl.when(wid >= NW // 2)
        def _():
            pltpu.sync_copy(rows_v, out_i.at[pl.ds(off - HALF, CH)])


@functools.cache
def _sc_gather():
    return pl.kernel(
        _gather_body,
        out_type=(
            jax.ShapeDtypeStruct((HALF, DIM), jnp.float32),
            jax.ShapeDtypeStruct((HALF, DIM), jnp.float32),
        ),
        mesh=plsc.VectorSubcoreMesh(
            core_axis_name="c", subcore_axis_name="s",
            num_cores=NC, num_subcores=NS,
        ),
        scratch_types=[
            pltpu.VMEM((BPW,), jnp.int32),
            pltpu.VMEM((CH, DIM), jnp.float32),
            pltpu.SemaphoreType.DMA,
        ],
        compiler_params=pltpu.CompilerParams(use_tc_tiling_on_sc=False),
    )


BB = 64


def _attn_body(h_ref, it_ref, ii_ref, ti_ref, hs_ref, mf_ref, a1_ref, a2_ref,
               out_ref):
    hs = hs_ref[...]
    mf = mf_ref[...]
    a1 = a1_ref[...][0:1, :].reshape(1, 1, DIM)
    a2 = a2_ref[...][0:1, :].reshape(1, 1, DIM)

    h = h_ref[...] * (ii_ref[...] != 0).astype(jnp.float32)[:, :, None]
    itm = (it_ref[...] * (ti_ref[...] != 0).astype(jnp.float32)[:, :, None]
           * mf[:, :, None])
    sess = jnp.sum(itm, axis=1) / jnp.sum(mf, axis=1)[:, None]

    att_n = jnp.sum(h * a1, axis=-1)
    att_n = jnp.where(att_n >= 0, att_n, ALPHA * att_n)
    w = jnp.where(hs > 0, hs * att_n[:, :, None], NEG)
    w = w - jnp.max(w, axis=1, keepdims=True)
    w = jnp.exp(w)
    w = w / jnp.sum(w, axis=1, keepdims=True)
    edge = lax.dot_general(w, h, (((1,), (1,)), ((0,), (0,))),
                           preferred_element_type=jnp.float32)

    att_e = jnp.sum(edge * a2, axis=-1)
    att_e = jnp.where(att_e >= 0, att_e, ALPHA * att_e)
    w2 = jnp.where(hs > 0, hs * att_e[:, None, :], NEG)
    w2 = w2 - jnp.max(w2, axis=2, keepdims=True)
    w2 = jnp.exp(w2)
    w2 = w2 / jnp.sum(w2, axis=2, keepdims=True)
    out = lax.dot_general(w2, edge, (((2,), (1,)), ((0,), (0,))),
                          preferred_element_type=jnp.float32)
    out_ref[...] = out + sess[:, None, :]


def _attention(h, itm, inputs, item, Hs, maskf, a1b, a2b):
    grid = (B // BB,)
    return pl.pallas_call(
        _attn_body,
        grid=grid,
        in_specs=[
            pl.BlockSpec((BB, L, DIM), lambda i: (i, 0, 0)),
            pl.BlockSpec((BB, L, DIM), lambda i: (i, 0, 0)),
            pl.BlockSpec((BB, L), lambda i: (i, 0)),
            pl.BlockSpec((BB, L), lambda i: (i, 0)),
            pl.BlockSpec((BB, L, L), lambda i: (i, 0, 0)),
            pl.BlockSpec((BB, L), lambda i: (i, 0)),
            pl.BlockSpec((8, DIM), lambda i: (0, 0)),
            pl.BlockSpec((8, DIM), lambda i: (0, 0)),
        ],
        out_specs=pl.BlockSpec((BB, L, DIM), lambda i: (i, 0, 0)),
        out_shape=jax.ShapeDtypeStruct((B, L, DIM), jnp.float32),
    )(h, itm, inputs, item, Hs, maskf, a1b, a2b)


def kernel(inputs, Hs, mask_item, item, embedding, a1, a2):
    inputs = inputs.astype(jnp.int32)
    item = item.astype(jnp.int32)
    idx = jnp.concatenate([inputs.reshape(-1), item.reshape(-1)])
    safe = jnp.maximum(idx - 1, 0)
    h_rows, item_rows = _sc_gather()(embedding, safe)
    h_rows = h_rows.reshape(B, L, DIM)
    item_rows = item_rows.reshape(B, L, DIM)

    maskf = mask_item.astype(jnp.float32)
    a1b = jnp.broadcast_to(a1.reshape(1, DIM), (8, DIM))
    a2b = jnp.broadcast_to(a2.reshape(1, DIM), (8, DIM))
    h_local = _attention(h_rows, item_rows, inputs, item, Hs, maskf, a1b, a2b)

    item_embeddings = jnp.concatenate(
        [jnp.zeros((1, DIM), dtype=embedding.dtype), embedding], axis=0)
    return (h_local, item_embeddings)


# speedup vs baseline: 1.2248x; 1.0529x over previous
import functools

import jax
import jax.numpy as jnp
from jax import lax
from jax.experimental import pallas as pl
from jax.experimental.pallas import tpu as pltpu
from jax.experimental.pallas import tpu_sc as plsc

DIM = 64
DPAD = 128
B = 1024
L = 50
ALPHA = 0.2
NEG = -1e9

NC = 2
NS = 16
NW = NC * NS
NIDX = 2 * B * L
BPW = NIDX // NW
CH = 800
HALF = B * L


def _gather_body(table, idx, out_h, out_i, idx_v, rows_v, sem):
    wid = lax.axis_index("s") * NC + lax.axis_index("c")
    base = wid * BPW
    pltpu.sync_copy(idx.at[pl.ds(base, BPW)], idx_v)
    for c in range(BPW // CH):
        pltpu.async_copy(table.at[idx_v.at[pl.ds(c * CH, CH)]], rows_v, sem).wait()
        off = base + c * CH

        @pl.when(wid < NW // 2)
        def _():
            pltpu.sync_copy(rows_v, out_h.at[pl.ds(off, CH)])

        @pl.when(wid >= NW // 2)
        def _():
            pltpu.sync_copy(rows_v, out_i.at[pl.ds(off - HALF, CH)])


@functools.cache
def _sc_gather():
    return pl.kernel(
        _gather_body,
        out_type=(
            jax.ShapeDtypeStruct((HALF, DPAD), jnp.float32),
            jax.ShapeDtypeStruct((HALF, DPAD), jnp.float32),
        ),
        mesh=plsc.VectorSubcoreMesh(
            core_axis_name="c", subcore_axis_name="s",
            num_cores=NC, num_subcores=NS,
        ),
        scratch_types=[
            pltpu.VMEM((BPW,), jnp.int32),
            pltpu.VMEM((CH, DPAD), jnp.float32),
            pltpu.SemaphoreType.DMA,
        ],
        compiler_params=pltpu.CompilerParams(use_tc_tiling_on_sc=True),
    )


BB = 64


def _attn_body(h_ref, it_ref, hs_ref, mf_ref, a1_ref, a2_ref, out_ref):
    hs = hs_ref[...]
    mf = mf_ref[...]
    a1 = a1_ref[...][0:1, :].reshape(1, 1, DPAD)
    a2 = a2_ref[...][0:1, :].reshape(1, 1, DPAD)

    h = h_ref[...]
    itm = it_ref[...] * mf[:, :, None]
    sess = jnp.sum(itm, axis=1) / jnp.sum(mf, axis=1)[:, None]

    att_n = jnp.sum(h * a1, axis=-1)
    att_n = jnp.where(att_n >= 0, att_n, ALPHA * att_n)
    w = jnp.where(hs > 0, hs * att_n[:, :, None], NEG)
    w = w - jnp.max(w, axis=1, keepdims=True)
    w = jnp.exp(w)
    w = w / jnp.sum(w, axis=1, keepdims=True)
    edge = lax.dot_general(w, h, (((1,), (1,)), ((0,), (0,))),
                           preferred_element_type=jnp.float32)

    att_e = jnp.sum(edge * a2, axis=-1)
    att_e = jnp.where(att_e >= 0, att_e, ALPHA * att_e)
    w2 = jnp.where(hs > 0, hs * att_e[:, None, :], NEG)
    w2 = w2 - jnp.max(w2, axis=2, keepdims=True)
    w2 = jnp.exp(w2)
    w2 = w2 / jnp.sum(w2, axis=2, keepdims=True)
    out = lax.dot_general(w2, edge, (((2,), (1,)), ((0,), (0,))),
                          preferred_element_type=jnp.float32)
    out = out + sess[:, None, :]
    out_ref[...] = out[:, :, :DIM]


def _attention(h, itm, Hs, maskf, a1b, a2b):
    grid = (B // BB,)
    return pl.pallas_call(
        _attn_body,
        grid=grid,
        in_specs=[
            pl.BlockSpec((BB, L, DPAD), lambda i: (i, 0, 0)),
            pl.BlockSpec((BB, L, DPAD), lambda i: (i, 0, 0)),
            pl.BlockSpec((BB, L, L), lambda i: (i, 0, 0)),
            pl.BlockSpec((BB, L), lambda i: (i, 0)),
            pl.BlockSpec((8, DPAD), lambda i: (0, 0)),
            pl.BlockSpec((8, DPAD), lambda i: (0, 0)),
        ],
        out_specs=pl.BlockSpec((BB, L, DIM), lambda i: (i, 0, 0)),
        out_shape=jax.ShapeDtypeStruct((B, L, DIM), jnp.float32),
    )(h, itm, Hs, maskf, a1b, a2b)


def kernel(inputs, Hs, mask_item, item, embedding, a1, a2):
    inputs = inputs.astype(jnp.int32)
    item = item.astype(jnp.int32)
    idx = jnp.concatenate([inputs.reshape(-1), item.reshape(-1)])

    table = jnp.pad(embedding, ((1, 0), (0, DPAD - DIM)))
    h_rows, item_rows = _sc_gather()(table, idx)
    h_rows = h_rows.reshape(B, L, DPAD)
    item_rows = item_rows.reshape(B, L, DPAD)

    maskf = mask_item.astype(jnp.float32)
    a1b = jnp.pad(a1.reshape(1, DIM), ((0, 7), (0, DPAD - DIM)))
    a2b = jnp.pad(a2.reshape(1, DIM), ((0, 7), (0, DPAD - DIM)))
    h_local = _attention(h_rows, item_rows, Hs, maskf, a1b, a2b)

    item_embeddings = jnp.pad(embedding, ((1, 0), (0, 0)))
    return (h_local, item_embeddings)
